# baseline (speedup 1.0000x reference)
import functools

import jax
import jax.numpy as jnp
from jax import lax
from jax.experimental import pallas as pl
from jax.experimental.pallas import tpu as pltpu
from jax.experimental.pallas import tpu_sc as plsc

_NC, _NS, _L = 2, 16, 16
_NW = _NC * _NS
_CH = 32
_UNROLL = 8


def _sc_body(batch, seq_len, d_model, x_hbm, pos_hbm, pe_hbm, out_hbm,
             pos_v, pe_v, xo_v):
    wid = lax.axis_index("s") * _NC + lax.axis_index("c")
    rows_per_w = seq_len // _NW
    n_chunks = rows_per_w // _CH
    chunk_words = _CH * d_model
    n_groups = chunk_words // (_L * _UNROLL)
    base = wid * rows_per_w * d_model

    for ci in range(n_chunks):
        off = base + ci * chunk_words
        pltpu.sync_copy(pos_hbm.at[pl.ds(off, chunk_words)], pos_v)
        pltpu.sync_copy(pe_hbm.at[pl.ds(off, chunk_words)], pe_v)

        def blend(g, _):
            for u in range(_UNROLL):
                s = pl.ds((g * _UNROLL + u) * _L, _L)
                pos_v[s] = 0.7 * pos_v[s] + 0.3 * pe_v[s]
            return 0

        lax.fori_loop(0, n_groups, blend, 0)

        for b in range(batch):
            xoff = b * seq_len * d_model + off
            pltpu.sync_copy(x_hbm.at[pl.ds(xoff, chunk_words)], xo_v)

            def add(g, _):
                for u in range(_UNROLL):
                    s = pl.ds((g * _UNROLL + u) * _L, _L)
                    xo_v[s] = xo_v[s] + pos_v[s]
                return 0

            lax.fori_loop(0, n_groups, add, 0)
            pltpu.sync_copy(xo_v, out_hbm.at[pl.ds(xoff, chunk_words)])


def kernel(x, pos_table, pe):
    batch, seq_len, d_model = x.shape
    chunk_words = _CH * d_model
    mesh = plsc.VectorSubcoreMesh(core_axis_name="c", subcore_axis_name="s")
    sc_call = pl.kernel(
        functools.partial(_sc_body, batch, seq_len, d_model),
        out_type=jax.ShapeDtypeStruct((batch * seq_len * d_model,), x.dtype),
        mesh=mesh,
        scratch_types=[
            pltpu.VMEM((chunk_words,), jnp.float32),
            pltpu.VMEM((chunk_words,), jnp.float32),
            pltpu.VMEM((chunk_words,), jnp.float32),
        ],
    )
    out = sc_call(
        x.reshape(-1),
        pos_table[:seq_len].reshape(-1),
        pe[:seq_len].reshape(-1),
    )
    return out.reshape(x.shape)


# speedup vs baseline: 1.1543x; 1.1543x over previous
import functools

import jax
import jax.numpy as jnp
from jax import lax
from jax.experimental import pallas as pl
from jax.experimental.pallas import tpu as pltpu
from jax.experimental.pallas import tpu_sc as plsc

_NC, _NS, _L = 2, 16, 16
_NW = _NC * _NS
_CH = 16


def _blend(dst, src, n_vec):
    @plsc.parallel_loop(0, n_vec, 1, unroll=8)
    def _(i):
        s = pl.ds(i * _L, _L)
        dst[s] = 0.7 * dst[s] + 0.3 * src[s]


def _add(dst, src, n_vec):
    @plsc.parallel_loop(0, n_vec, 1, unroll=8)
    def _(i):
        s = pl.ds(i * _L, _L)
        dst[s] = dst[s] + src[s]


def _sc_body(batch, seq_len, d_model, x_hbm, pos_hbm, pe_hbm, out_hbm,
             pos_v, pe_v, xo_v, stab, sx, sst):
    wid = lax.axis_index("s") * _NC + lax.axis_index("c")
    rows_per_w = seq_len // _NW
    n_chunks = rows_per_w // _CH
    cw = _CH * d_model
    n_vec = cw // _L
    base = wid * rows_per_w * d_model

    def tab_off(ci):
        return base + ci * cw

    def x_off(ci, b):
        return b * seq_len * d_model + tab_off(ci)

    steps = [(ci, b) for ci in range(n_chunks) for b in range(batch)]
    n_steps = len(steps)

    tab_d = [
        pltpu.async_copy(pos_hbm.at[pl.ds(tab_off(0), cw)], pos_v[0], stab),
        pltpu.async_copy(pe_hbm.at[pl.ds(tab_off(0), cw)], pe_v[0], stab),
    ]
    xl_d = pltpu.async_copy(
        x_hbm.at[pl.ds(x_off(0, 0), cw)], xo_v[0], sx[0])
    st_d = [None, None]

    for k, (ci, b) in enumerate(steps):
        p = k % 2
        cp = ci % 2
        if b == 0:
            tab_d[0].wait()
            tab_d[1].wait()
            if ci + 1 < n_chunks:
                ncp = (ci + 1) % 2
                tab_d = [
                    pltpu.async_copy(
                        pos_hbm.at[pl.ds(tab_off(ci + 1), cw)],
                        pos_v[ncp], stab),
                    pltpu.async_copy(
                        pe_hbm.at[pl.ds(tab_off(ci + 1), cw)],
                        pe_v[ncp], stab),
                ]
            _blend(pos_v[cp], pe_v[cp], n_vec)
        xl_d.wait()
        if k + 1 < n_steps:
            np_ = (k + 1) % 2
            if st_d[np_] is not None:
                st_d[np_].wait()
                st_d[np_] = None
            nci, nb = steps[k + 1]
            xl_d = pltpu.async_copy(
                x_hbm.at[pl.ds(x_off(nci, nb), cw)], xo_v[np_], sx[np_])
        _add(xo_v[p], pos_v[cp], n_vec)
        st_d[p] = pltpu.async_copy(
            xo_v[p], out_hbm.at[pl.ds(x_off(ci, b), cw)], sst[p])

    for d in st_d:
        if d is not None:
            d.wait()


def kernel(x, pos_table, pe):
    batch, seq_len, d_model = x.shape
    cw = _CH * d_model
    mesh = plsc.VectorSubcoreMesh(core_axis_name="c", subcore_axis_name="s")
    sc_call = pl.kernel(
        functools.partial(_sc_body, batch, seq_len, d_model),
        out_type=jax.ShapeDtypeStruct((batch * seq_len * d_model,), x.dtype),
        mesh=mesh,
        scratch_types=[
            [pltpu.VMEM((cw,), jnp.float32) for _ in range(2)],
            [pltpu.VMEM((cw,), jnp.float32) for _ in range(2)],
            [pltpu.VMEM((cw,), jnp.float32) for _ in range(2)],
            pltpu.SemaphoreType.DMA,
            [pltpu.SemaphoreType.DMA for _ in range(2)],
            [pltpu.SemaphoreType.DMA for _ in range(2)],
        ],
    )
    out = sc_call(
        x.reshape(-1),
        pos_table[:seq_len].reshape(-1),
        pe[:seq_len].reshape(-1),
    )
    return out.reshape(x.shape)
